# baseline (speedup 1.0000x reference)
import functools

import jax
import jax.numpy as jnp
import numpy as np
from jax.experimental import pallas as pl

_NUM_BINS = 12
_ANGLE_PER_BIN = 2.0 * np.pi / _NUM_BINS
_TWO_PI = 2.0 * np.pi


def _rpn_kernel(x_ref, w1_ref, b1_ref, w2_ref, b2_ref,
                wh_ref, bh_ref, out_ref):
    x = x_ref[...]
    h = jnp.maximum(
        jax.lax.dot_general(w1_ref[...], x, (((0,), (1,)), ((), ())),
                            preferred_element_type=jnp.float32)
        + b1_ref[...], 0.0)
    h = jnp.maximum(
        jax.lax.dot_general(w2_ref[...], h, (((0,), (0,)), ((), ())),
                            preferred_element_type=jnp.float32)
        + b2_ref[...], 0.0)
    o = (jax.lax.dot_general(wh_ref[...], h, (((0,), (0,)), ((), ())),
                             preferred_element_type=jnp.float32)
         + bh_ref[...])

    obj = jax.nn.sigmoid(o[0:1, :])

    hcls = o[7:7 + _NUM_BINS, :]
    hd = o[7 + _NUM_BINS:7 + 2 * _NUM_BINS, :]
    mx = jnp.max(hcls, axis=0, keepdims=True)
    iota = jax.lax.broadcasted_iota(jnp.int32, hcls.shape, 0)
    idx = jnp.min(jnp.where(hcls == mx, iota, _NUM_BINS),
                  axis=0, keepdims=True)
    delta = jnp.sum(jnp.where(iota == idx, hd, 0.0), axis=0, keepdims=True)
    ang = jnp.mod(idx.astype(jnp.float32) * _ANGLE_PER_BIN + delta, _TWO_PI)

    out_ref[...] = jnp.concatenate([obj, ang, o[1:7, :]], axis=0)


@functools.partial(jax.jit, static_argnames=())
def kernel(voted_xyz, voted_features, W1, b1, W2, b2, W_obj, b_obj,
           W_box, b_box, W_hcls, b_hcls, W_hd, b_hd):
    B, N, C = voted_features.shape
    H = W1.shape[1]
    M = B * N
    T = 8192
    grid = (M // T,)

    x = voted_features.reshape(M, C)
    wh = jnp.concatenate(
        [W_obj, W_box, W_hcls, W_hd,
         jnp.zeros((H, 1), dtype=W_obj.dtype)], axis=1)
    bh = jnp.concatenate(
        [b_obj, b_box, b_hcls, b_hd,
         jnp.zeros((1,), dtype=b_obj.dtype)], axis=0)

    out = pl.pallas_call(
        _rpn_kernel,
        grid=grid,
        in_specs=[
            pl.BlockSpec((T, C), lambda i: (i, 0)),
            pl.BlockSpec((C, H), lambda i: (0, 0)),
            pl.BlockSpec((H, 1), lambda i: (0, 0)),
            pl.BlockSpec((H, H), lambda i: (0, 0)),
            pl.BlockSpec((H, 1), lambda i: (0, 0)),
            pl.BlockSpec((H, 32), lambda i: (0, 0)),
            pl.BlockSpec((32, 1), lambda i: (0, 0)),
        ],
        out_specs=pl.BlockSpec((8, T), lambda i: (0, i)),
        out_shape=jax.ShapeDtypeStruct((8, M), jnp.float32),
    )(x, W1, b1.reshape(H, 1), W2, b2.reshape(H, 1), wh, bh.reshape(32, 1))

    obj = out[0].reshape(B, N)
    ang = out[1].reshape(B, N)
    d = out[2:8].T
    xyz = voted_xyz.reshape(M, 3)
    boxes = jnp.concatenate([xyz - d[:, 0:3], xyz + d[:, 3:6]],
                            axis=-1).reshape(B, N, 6)
    return (obj, boxes, ang)
